# baseline (speedup 1.0000x reference)
import functools

import jax
import jax.numpy as jnp
from jax import lax
from jax.experimental import pallas as pl
from jax.experimental.pallas import tpu as pltpu
from jax.experimental.pallas import tpu_sc as plsc

N_NODES = 10000
N_EDGES = 320000
NC, NS, L = 2, 16, 16
NW = NC * NS
B = 80
RPW = N_EDGES // (NW * B)
EPW = RPW * B
NPAD = 10240
SP = NPAD // NS
F32 = jnp.float32

_MESH = plsc.VectorSubcoreMesh(core_axis_name="c", subcore_axis_name="s")
_SC_PARAMS = pltpu.CompilerParams(needs_layout_passes=False,
                                  use_tc_tiling_on_sc=False)


def _edge_pass_body(ei_h, w_h, ts_h, b_h, p_h, z_h, e_out, agg_out,
                    src_v, dst_v, w_v, e_v, ts_a, ts_b, b_a, b_b,
                    msg_a, msg_b, mtx, pbuf, agg_sh,
                    g1a, g1b, g2a, g2b, sca, scb):
    cid = lax.axis_index("c")
    sid = lax.axis_index("s")
    wid = sid * NC + cid

    pltpu.sync_copy(z_h, agg_sh.at[pl.ds(sid * SP, SP)])
    plsc.subcore_barrier()

    pltpu.sync_copy(ei_h.at[0, wid], src_v)
    pltpu.sync_copy(ei_h.at[1, wid], dst_v)
    pltpu.sync_copy(w_h.at[pl.ds(wid * EPW, EPW)], w_v)
    pltpu.sync_copy(p_h, pbuf)
    wc0 = pbuf[pl.ds(0, L)]
    wc1 = pbuf[pl.ds(L, L)]
    wr0 = pbuf[pl.ds(2 * L, L)]
    wr1 = pbuf[pl.ds(3 * L, L)]
    brv = pbuf[pl.ds(4 * L, L)]
    br = brv[0]
    iot = lax.iota(jnp.int32, L)

    def _g(bi, ts_buf, b_buf, sem_ts, sem_b):
        return (pltpu.make_async_copy(ts_h.at[src_v.at[bi]], ts_buf, sem_ts),
                pltpu.make_async_copy(b_h.at[dst_v.at[bi]], b_buf, sem_b))

    def _sc(bi, msg, sem):
        return pltpu.make_async_copy(msg, agg_sh.at[dst_v.at[bi]], sem)

    def _compute(bi, ts_rows, b_rows, msg):
        for g in range(B // L):
            wv = w_v[pl.ds(bi * B + g * L, L)]
            for j in range(L):
                e = g * L + j
                w = wv[j]
                u0 = ts_rows[e, pl.ds(2 * L, L)] + b_rows[e, pl.ds(0, L)] + w * wc0
                u1 = ts_rows[e, pl.ds(3 * L, L)] + b_rows[e, pl.ds(L, L)] + w * wc1
                mtx[j] = (jnp.maximum(u0, 0.0) * wr0
                          + jnp.maximum(u1, 0.0) * wr1)
                msg[e, pl.ds(0, L)] = w * ts_rows[e, pl.ds(0, L)]
                msg[e, pl.ds(L, L)] = w * ts_rows[e, pl.ds(L, L)]
            tot = jnp.full((L,), br, F32)
            for c in range(L):
                tot = tot + plsc.load_gather(
                    mtx, [iot, jnp.full((L,), c, jnp.int32)])
            e_v[pl.ds(bi * B + g * L, L)] = tot

    for cp in _g(0, ts_a, b_a, g1a, g2a):
        cp.start()
    for cp in _g(1, ts_b, b_b, g1b, g2b):
        cp.start()

    for cp in _g(0, ts_a, b_a, g1a, g2a):
        cp.wait()
    _compute(0, ts_a, b_a, msg_a)
    _sc(0, msg_a, sca).start(add=True)
    for cp in _g(2, ts_a, b_a, g1a, g2a):
        cp.start()

    for cp in _g(1, ts_b, b_b, g1b, g2b):
        cp.wait()
    _compute(1, ts_b, b_b, msg_b)
    _sc(1, msg_b, scb).start(add=True)
    for cp in _g(3, ts_b, b_b, g1b, g2b):
        cp.start()

    def body(i, carry):
        bi0 = 2 * i + 2
        bi1 = bi0 + 1
        for cp in _g(bi0, ts_a, b_a, g1a, g2a):
            cp.wait()
        _sc(bi0 - 2, msg_a, sca).wait()
        _compute(bi0, ts_a, b_a, msg_a)
        _sc(bi0, msg_a, sca).start(add=True)
        for cp in _g(bi0 + 2, ts_a, b_a, g1a, g2a):
            cp.start()
        for cp in _g(bi1, ts_b, b_b, g1b, g2b):
            cp.wait()
        _sc(bi1 - 2, msg_b, scb).wait()
        _compute(bi1, ts_b, b_b, msg_b)
        _sc(bi1, msg_b, scb).start(add=True)
        for cp in _g(bi1 + 2, ts_b, b_b, g1b, g2b):
            cp.start()
        return carry

    lax.fori_loop(0, (RPW - 5) // 2, body, 0)

    for cp in _g(RPW - 3, ts_a, b_a, g1a, g2a):
        cp.wait()
    _sc(RPW - 5, msg_a, sca).wait()
    _compute(RPW - 3, ts_a, b_a, msg_a)
    _sc(RPW - 3, msg_a, sca).start(add=True)
    for cp in _g(RPW - 1, ts_a, b_a, g1a, g2a):
        cp.start()

    for cp in _g(RPW - 2, ts_b, b_b, g1b, g2b):
        cp.wait()
    _sc(RPW - 4, msg_b, scb).wait()
    _compute(RPW - 2, ts_b, b_b, msg_b)
    _sc(RPW - 2, msg_b, scb).start(add=True)

    for cp in _g(RPW - 1, ts_a, b_a, g1a, g2a):
        cp.wait()
    _sc(RPW - 3, msg_a, sca).wait()
    _compute(RPW - 1, ts_a, b_a, msg_a)
    _sc(RPW - 1, msg_a, sca).start(add=True)

    _sc(RPW - 1, msg_a, sca).wait()
    _sc(RPW - 2, msg_b, scb).wait()

    pltpu.sync_copy(e_v, e_out.at[pl.ds(wid * EPW, EPW)])
    plsc.subcore_barrier()
    pltpu.sync_copy(agg_sh.at[pl.ds(sid * SP, SP)],
                    agg_out.at[cid, pl.ds(sid * SP, SP)])


_sc_edge_pass = functools.partial(
    pl.kernel,
    out_type=[jax.ShapeDtypeStruct((N_EDGES,), F32),
              jax.ShapeDtypeStruct((NC, NPAD, 2 * L), F32)],
    mesh=_MESH,
    compiler_params=_SC_PARAMS,
    scratch_types=[
        pltpu.VMEM((RPW, B), jnp.int32),
        pltpu.VMEM((RPW, B), jnp.int32),
        pltpu.VMEM((EPW,), F32),
        pltpu.VMEM((EPW,), F32),
        pltpu.VMEM((B, 4 * L), F32),
        pltpu.VMEM((B, 4 * L), F32),
        pltpu.VMEM((B, 2 * L), F32),
        pltpu.VMEM((B, 2 * L), F32),
        pltpu.VMEM((B, 2 * L), F32),
        pltpu.VMEM((B, 2 * L), F32),
        pltpu.VMEM((L, L), F32),
        pltpu.VMEM((5 * L,), F32),
        pltpu.VMEM_SHARED((NPAD, 2 * L), F32),
        pltpu.SemaphoreType.DMA,
        pltpu.SemaphoreType.DMA,
        pltpu.SemaphoreType.DMA,
        pltpu.SemaphoreType.DMA,
        pltpu.SemaphoreType.DMA,
        pltpu.SemaphoreType.DMA,
    ],
)(_edge_pass_body)


def _agg_pass_body(ei_h, w_h, ts_h, z_h, agg_out,
                   src_v, dst_v, w_v, ts_a, ts_b, msg_a, msg_b, agg_sh,
                   g1a, g1b, sca, scb):
    cid = lax.axis_index("c")
    sid = lax.axis_index("s")
    wid = sid * NC + cid

    pltpu.sync_copy(z_h, agg_sh.at[pl.ds(sid * SP, SP)])
    plsc.subcore_barrier()

    pltpu.sync_copy(ei_h.at[0, wid], src_v)
    pltpu.sync_copy(ei_h.at[1, wid], dst_v)
    pltpu.sync_copy(w_h.at[pl.ds(wid * EPW, EPW)], w_v)

    def _g(bi, ts_buf, sem):
        return pltpu.make_async_copy(ts_h.at[src_v.at[bi]], ts_buf, sem)

    def _sc(bi, msg, sem):
        return pltpu.make_async_copy(msg, agg_sh.at[dst_v.at[bi]], sem)

    def _compute(bi, ts_rows, msg):
        for g in range(B // L):
            wv = w_v[pl.ds(bi * B + g * L, L)]
            for j in range(L):
                e = g * L + j
                w = wv[j]
                for k in range(4):
                    msg[e, pl.ds(k * L, L)] = w * ts_rows[e, pl.ds(k * L, L)]

    _g(0, ts_a, g1a).start()
    _g(1, ts_b, g1b).start()

    _g(0, ts_a, g1a).wait()
    _compute(0, ts_a, msg_a)
    _sc(0, msg_a, sca).start(add=True)
    _g(2, ts_a, g1a).start()

    _g(1, ts_b, g1b).wait()
    _compute(1, ts_b, msg_b)
    _sc(1, msg_b, scb).start(add=True)
    _g(3, ts_b, g1b).start()

    def body(i, carry):
        bi0 = 2 * i + 2
        bi1 = bi0 + 1
        _g(bi0, ts_a, g1a).wait()
        _sc(bi0 - 2, msg_a, sca).wait()
        _compute(bi0, ts_a, msg_a)
        _sc(bi0, msg_a, sca).start(add=True)
        _g(bi0 + 2, ts_a, g1a).start()
        _g(bi1, ts_b, g1b).wait()
        _sc(bi1 - 2, msg_b, scb).wait()
        _compute(bi1, ts_b, msg_b)
        _sc(bi1, msg_b, scb).start(add=True)
        _g(bi1 + 2, ts_b, g1b).start()
        return carry

    lax.fori_loop(0, (RPW - 5) // 2, body, 0)

    _g(RPW - 3, ts_a, g1a).wait()
    _sc(RPW - 5, msg_a, sca).wait()
    _compute(RPW - 3, ts_a, msg_a)
    _sc(RPW - 3, msg_a, sca).start(add=True)
    _g(RPW - 1, ts_a, g1a).start()

    _g(RPW - 2, ts_b, g1b).wait()
    _sc(RPW - 4, msg_b, scb).wait()
    _compute(RPW - 2, ts_b, msg_b)
    _sc(RPW - 2, msg_b, scb).start(add=True)

    _g(RPW - 1, ts_a, g1a).wait()
    _sc(RPW - 3, msg_a, sca).wait()
    _compute(RPW - 1, ts_a, msg_a)
    _sc(RPW - 1, msg_a, sca).start(add=True)

    _sc(RPW - 1, msg_a, sca).wait()
    _sc(RPW - 2, msg_b, scb).wait()

    plsc.subcore_barrier()
    pltpu.sync_copy(agg_sh.at[pl.ds(sid * SP, SP)],
                    agg_out.at[cid, pl.ds(sid * SP, SP)])


_sc_agg_pass = functools.partial(
    pl.kernel,
    out_type=[jax.ShapeDtypeStruct((NC, NPAD, 4 * L), F32)],
    mesh=_MESH,
    compiler_params=_SC_PARAMS,
    scratch_types=[
        pltpu.VMEM((RPW, B), jnp.int32),
        pltpu.VMEM((RPW, B), jnp.int32),
        pltpu.VMEM((EPW,), F32),
        pltpu.VMEM((B, 4 * L), F32),
        pltpu.VMEM((B, 4 * L), F32),
        pltpu.VMEM((B, 4 * L), F32),
        pltpu.VMEM((B, 4 * L), F32),
        pltpu.VMEM_SHARED((NPAD, 4 * L), F32),
        pltpu.SemaphoreType.DMA,
        pltpu.SemaphoreType.DMA,
        pltpu.SemaphoreType.DMA,
        pltpu.SemaphoreType.DMA,
    ],
)(_agg_pass_body)


def _tc0_body(x_ref, w_ref, bf_ref, ts_ref, b_ref, r_ref):
    y = jnp.dot(x_ref[...], w_ref[...], preferred_element_type=F32)
    ts_ref[...] = y[:, 0:64]
    b_ref[...] = y[:, 64:96] + bf_ref[...]
    r_ref[...] = y[:, 96:128]


def _tc1_body(agg_ref, r_ref, brel_ref, w_ref, bf_ref, ts_ref, b_ref, r2_ref):
    agg = agg_ref[0, :N_NODES] + agg_ref[1, :N_NODES]
    x1 = jnp.maximum(agg + brel_ref[...] + r_ref[...], 0.0)
    y = jnp.dot(x1, w_ref[...], preferred_element_type=F32)
    ts_ref[...] = jnp.concatenate([x1, y[:, 0:32]], axis=1)
    b_ref[...] = y[:, 32:64] + bf_ref[...]
    r2_ref[...] = y[:, 64:128]


def _tc2_body(agg_ref, r_ref, wrel_ref, brel_ref, wroot_ref, ts_ref, r3_ref):
    a = agg_ref[0, :N_NODES] + agg_ref[1, :N_NODES]
    x2 = jnp.maximum(jnp.dot(a, wrel_ref[...], preferred_element_type=F32)
                     + brel_ref[...] + r_ref[...], 0.0)
    ts_ref[...] = x2
    r3_ref[...] = jnp.dot(x2, wroot_ref[...], preferred_element_type=F32)


def _tc3_body(agg_ref, r_ref, wrel_ref, brel_ref, out_ref):
    a = agg_ref[0, :N_NODES] + agg_ref[1, :N_NODES]
    out_ref[...] = jnp.maximum(
        jnp.dot(a, wrel_ref[...], preferred_element_type=F32)
        + brel_ref[...] + r_ref[...], 0.0)


def kernel(x, edge_index, edge_attr, Wrel1, brel1, Wroot1, Wrel2, brel2,
           Wroot2, Wrel3, brel3, Wroot3, Wf1, bf1, Wr1, br1, Wf2, bf2, Wr2,
           br2):
    ei = edge_index.reshape(2, NW, RPW, B)
    z32 = jnp.zeros((SP, 32), F32)
    z64 = jnp.zeros((SP, 64), F32)

    W0 = jnp.concatenate(
        [Wrel1.T, Wf1[:, :128].T, Wf1[:, 128:256].T, Wroot1.T], axis=1)
    ts1, b1p, r1 = pl.pallas_call(
        _tc0_body,
        out_shape=[jax.ShapeDtypeStruct((N_NODES, 64), F32),
                   jax.ShapeDtypeStruct((N_NODES, 32), F32),
                   jax.ShapeDtypeStruct((N_NODES, 32), F32)],
    )(x, W0, bf1.reshape(1, 32))

    p1 = jnp.concatenate([Wf1[:, 256], Wr1[0], br1, jnp.zeros((15,), F32)])
    e1, agg1 = _sc_edge_pass(ei, edge_attr, ts1, b1p, p1, z32)

    W1 = jnp.concatenate([Wf2[:, :32].T, Wf2[:, 32:64].T, Wroot2.T], axis=1)
    ts2, b2p, r2 = pl.pallas_call(
        _tc1_body,
        out_shape=[jax.ShapeDtypeStruct((N_NODES, 64), F32),
                   jax.ShapeDtypeStruct((N_NODES, 32), F32),
                   jax.ShapeDtypeStruct((N_NODES, 64), F32)],
    )(agg1, r1, brel1.reshape(1, 32), W1, bf2.reshape(1, 32))

    p2 = jnp.concatenate([Wf2[:, 64], Wr2[0], br2, jnp.zeros((15,), F32)])
    e2, agg2 = _sc_edge_pass(ei, e1, ts2, b2p, p2, z32)

    ts3, r3 = pl.pallas_call(
        _tc2_body,
        out_shape=[jax.ShapeDtypeStruct((N_NODES, 64), F32),
                   jax.ShapeDtypeStruct((N_NODES, 128), F32)],
    )(agg2, r2, Wrel2.T, brel2.reshape(1, 64), Wroot3.T)

    (agg3,) = _sc_agg_pass(ei, e2, ts3, z64)

    x3 = pl.pallas_call(
        _tc3_body,
        out_shape=jax.ShapeDtypeStruct((N_NODES, 128), F32),
    )(agg3, r3, Wrel3.T, brel3.reshape(1, 128))
    return x3


# speedup vs baseline: 1.1695x; 1.1695x over previous
import functools

import jax
import jax.numpy as jnp
from jax import lax
from jax.experimental import pallas as pl
from jax.experimental.pallas import tpu as pltpu
from jax.experimental.pallas import tpu_sc as plsc

N_NODES = 10000
N_EDGES = 320000
NC, NS, L = 2, 16, 16
NW = NC * NS
B = 80
RPW = N_EDGES // (NW * B)
NPAD = 10240
SP = NPAD // NS
F32 = jnp.float32

_MESH = plsc.VectorSubcoreMesh(core_axis_name="c", subcore_axis_name="s")
_SC_PARAMS = pltpu.CompilerParams(needs_layout_passes=False,
                                  use_tc_tiling_on_sc=False)


def _edge_pass_body(src_h, dst_h, w_h, ts_h, b_h, p_h, z_h, e_out, agg_out,
                    src_v, dst_v, w_v, e_v, ts_a, ts_b, b_a, b_b, msg_a,
                    msg_b, mtx, pbuf, agg_sh, s1a, s1b, s2a, s2b, sca, scb):
    cid = lax.axis_index("c")
    sid = lax.axis_index("s")
    wid = sid * NC + cid

    pltpu.sync_copy(z_h, agg_sh.at[pl.ds(sid * SP, SP)])
    plsc.subcore_barrier()

    pltpu.sync_copy(src_h.at[wid], src_v)
    pltpu.sync_copy(dst_h.at[wid], dst_v)
    pltpu.sync_copy(w_h.at[wid], w_v)
    pltpu.sync_copy(p_h, pbuf)
    wc0 = pbuf[pl.ds(0, L)]
    wc1 = pbuf[pl.ds(L, L)]
    wr0 = pbuf[pl.ds(2 * L, L)]
    wr1 = pbuf[pl.ds(3 * L, L)]
    brv = pbuf[pl.ds(4 * L, L)]
    br = brv[0]
    iot = lax.iota(jnp.int32, L)

    def _g(bi, ts_buf, b_buf, sem_ts, sem_b):
        return (pltpu.make_async_copy(ts_h.at[src_v.at[bi]], ts_buf, sem_ts),
                pltpu.make_async_copy(b_h.at[dst_v.at[bi]], b_buf, sem_b))

    def _scd(bi, msg, sem):
        return pltpu.make_async_copy(msg, agg_sh.at[dst_v.at[bi]], sem)

    def _compute(bi, ts_rows, b_rows, msg):
        for g in range(B // L):
            wv = w_v[bi, pl.ds(g * L, L)]
            for j in range(L):
                e = g * L + j
                w = wv[j]
                u0 = ts_rows[e, pl.ds(2 * L, L)] + b_rows[e, pl.ds(0, L)] + w * wc0
                u1 = ts_rows[e, pl.ds(3 * L, L)] + b_rows[e, pl.ds(L, L)] + w * wc1
                mtx[j] = (jnp.maximum(u0, 0.0) * wr0
                          + jnp.maximum(u1, 0.0) * wr1)
                msg[e, pl.ds(0, L)] = w * ts_rows[e, pl.ds(0, L)]
                msg[e, pl.ds(L, L)] = w * ts_rows[e, pl.ds(L, L)]
            tot = jnp.full((L,), br, F32)
            for c in range(L):
                tot = tot + plsc.load_gather(
                    mtx, [iot, jnp.full((L,), c, jnp.int32)])
            e_v[bi, pl.ds(g * L, L)] = tot

    pltpu.make_async_copy(z_h.at[pl.ds(0, B)], msg_a, sca).start()
    pltpu.make_async_copy(z_h.at[pl.ds(0, B)], msg_b, scb).start()

    for cp in _g(0, ts_a, b_a, s1a, s2a):
        cp.start()

    def body(i, carry):
        bi0 = 2 * i
        bi1 = bi0 + 1
        for cp in _g(bi1, ts_b, b_b, s1b, s2b):
            cp.start()
        for cp in _g(bi0, ts_a, b_a, s1a, s2a):
            cp.wait()
        _scd(bi0, msg_a, sca).wait()
        _compute(bi0, ts_a, b_a, msg_a)
        _scd(bi0, msg_a, sca).start(add=True)
        for cp in _g(bi0 + 2, ts_a, b_a, s1a, s2a):
            cp.start()
        for cp in _g(bi1, ts_b, b_b, s1b, s2b):
            cp.wait()
        _scd(bi1, msg_b, scb).wait()
        _compute(bi1, ts_b, b_b, msg_b)
        _scd(bi1, msg_b, scb).start(add=True)
        return carry

    lax.fori_loop(0, (RPW - 1) // 2, body, 0)
    for cp in _g(RPW - 1, ts_a, b_a, s1a, s2a):
        cp.wait()
    _scd(RPW - 1, msg_a, sca).wait()
    _compute(RPW - 1, ts_a, b_a, msg_a)
    _scd(RPW - 1, msg_a, sca).start(add=True)

    _scd(RPW - 1, msg_a, sca).wait()
    _scd(RPW - 2, msg_b, scb).wait()

    pltpu.sync_copy(e_v, e_out.at[wid])
    plsc.subcore_barrier()
    pltpu.sync_copy(agg_sh.at[pl.ds(sid * SP, SP)],
                    agg_out.at[cid, pl.ds(sid * SP, SP)])


_sc_edge_pass = functools.partial(
    pl.kernel,
    out_type=[jax.ShapeDtypeStruct((NW, RPW, B), F32),
              jax.ShapeDtypeStruct((NC, NPAD, 2 * L), F32)],
    mesh=_MESH,
    compiler_params=_SC_PARAMS,
    scratch_types=[
        pltpu.VMEM((RPW, B), jnp.int32),
        pltpu.VMEM((RPW, B), jnp.int32),
        pltpu.VMEM((RPW, B), F32),
        pltpu.VMEM((RPW, B), F32),
        pltpu.VMEM((B, 4 * L), F32),
        pltpu.VMEM((B, 4 * L), F32),
        pltpu.VMEM((B, 2 * L), F32),
        pltpu.VMEM((B, 2 * L), F32),
        pltpu.VMEM((B, 2 * L), F32),
        pltpu.VMEM((B, 2 * L), F32),
        pltpu.VMEM((L, L), F32),
        pltpu.VMEM((5 * L,), F32),
        pltpu.VMEM_SHARED((NPAD, 2 * L), F32),
        pltpu.SemaphoreType.DMA,
        pltpu.SemaphoreType.DMA,
        pltpu.SemaphoreType.DMA,
        pltpu.SemaphoreType.DMA,
        pltpu.SemaphoreType.DMA,
        pltpu.SemaphoreType.DMA,
    ],
)(_edge_pass_body)


def _agg_pass_body(src_h, dst_h, w_h, ts_h, z_h, agg_out,
                   src_v, dst_v, w_v, ts_a, ts_b, msg_a, msg_b, agg_sh,
                   s1a, s1b, sca, scb):
    cid = lax.axis_index("c")
    sid = lax.axis_index("s")
    wid = sid * NC + cid

    pltpu.sync_copy(z_h, agg_sh.at[pl.ds(sid * SP, SP)])
    plsc.subcore_barrier()

    pltpu.sync_copy(src_h.at[wid], src_v)
    pltpu.sync_copy(dst_h.at[wid], dst_v)
    pltpu.sync_copy(w_h.at[wid], w_v)

    def _g(bi, ts_buf, sem):
        return pltpu.make_async_copy(ts_h.at[src_v.at[bi]], ts_buf, sem)

    def _scd(bi, msg, sem):
        return pltpu.make_async_copy(msg, agg_sh.at[dst_v.at[bi]], sem)

    def _compute(bi, ts_rows, msg):
        for g in range(B // L):
            wv = w_v[bi, pl.ds(g * L, L)]
            for j in range(L):
                e = g * L + j
                w = wv[j]
                for k in range(4):
                    msg[e, pl.ds(k * L, L)] = w * ts_rows[e, pl.ds(k * L, L)]

    pltpu.make_async_copy(z_h.at[pl.ds(0, B)], msg_a, sca).start()
    pltpu.make_async_copy(z_h.at[pl.ds(0, B)], msg_b, scb).start()

    _g(0, ts_a, s1a).start()

    def body(i, carry):
        bi0 = 2 * i
        bi1 = bi0 + 1
        _g(bi1, ts_b, s1b).start()
        _g(bi0, ts_a, s1a).wait()
        _scd(bi0, msg_a, sca).wait()
        _compute(bi0, ts_a, msg_a)
        _scd(bi0, msg_a, sca).start(add=True)
        _g(bi0 + 2, ts_a, s1a).start()
        _g(bi1, ts_b, s1b).wait()
        _scd(bi1, msg_b, scb).wait()
        _compute(bi1, ts_b, msg_b)
        _scd(bi1, msg_b, scb).start(add=True)
        return carry

    lax.fori_loop(0, (RPW - 1) // 2, body, 0)
    _g(RPW - 1, ts_a, s1a).wait()
    _scd(RPW - 1, msg_a, sca).wait()
    _compute(RPW - 1, ts_a, msg_a)
    _scd(RPW - 1, msg_a, sca).start(add=True)

    _scd(RPW - 1, msg_a, sca).wait()
    _scd(RPW - 2, msg_b, scb).wait()

    plsc.subcore_barrier()
    pltpu.sync_copy(agg_sh.at[pl.ds(sid * SP, SP)],
                    agg_out.at[cid, pl.ds(sid * SP, SP)])


_sc_agg_pass = functools.partial(
    pl.kernel,
    out_type=[jax.ShapeDtypeStruct((NC, NPAD, 4 * L), F32)],
    mesh=_MESH,
    compiler_params=_SC_PARAMS,
    scratch_types=[
        pltpu.VMEM((RPW, B), jnp.int32),
        pltpu.VMEM((RPW, B), jnp.int32),
        pltpu.VMEM((RPW, B), F32),
        pltpu.VMEM((B, 4 * L), F32),
        pltpu.VMEM((B, 4 * L), F32),
        pltpu.VMEM((B, 4 * L), F32),
        pltpu.VMEM((B, 4 * L), F32),
        pltpu.VMEM_SHARED((NPAD, 4 * L), F32),
        pltpu.SemaphoreType.DMA,
        pltpu.SemaphoreType.DMA,
        pltpu.SemaphoreType.DMA,
        pltpu.SemaphoreType.DMA,
    ],
)(_agg_pass_body)


def _tc0_body(x_ref, w_ref, bf_ref, ts_ref, b_ref, r_ref):
    y = jnp.dot(x_ref[...], w_ref[...], preferred_element_type=F32)
    ts_ref[...] = y[:, 0:64]
    b_ref[...] = y[:, 64:96] + bf_ref[...]
    r_ref[...] = y[:, 96:128]


def _tc1_body(agg_ref, r_ref, brel_ref, w_ref, bf_ref, ts_ref, b_ref, r2_ref):
    agg = agg_ref[0, :N_NODES] + agg_ref[1, :N_NODES]
    x1 = jnp.maximum(agg + brel_ref[...] + r_ref[...], 0.0)
    y = jnp.dot(x1, w_ref[...], preferred_element_type=F32)
    ts_ref[...] = jnp.concatenate([x1, y[:, 0:32]], axis=1)
    b_ref[...] = y[:, 32:64] + bf_ref[...]
    r2_ref[...] = y[:, 64:128]


def _tc2_body(agg_ref, r_ref, wrel_ref, brel_ref, wroot_ref, ts_ref, r3_ref):
    a = agg_ref[0, :N_NODES] + agg_ref[1, :N_NODES]
    x2 = jnp.maximum(jnp.dot(a, wrel_ref[...], preferred_element_type=F32)
                     + brel_ref[...] + r_ref[...], 0.0)
    ts_ref[...] = x2
    r3_ref[...] = jnp.dot(x2, wroot_ref[...], preferred_element_type=F32)


def _tc3_body(agg_ref, r_ref, wrel_ref, brel_ref, out_ref):
    a = agg_ref[0, :N_NODES] + agg_ref[1, :N_NODES]
    out_ref[...] = jnp.maximum(
        jnp.dot(a, wrel_ref[...], preferred_element_type=F32)
        + brel_ref[...] + r_ref[...], 0.0)


def kernel(x, edge_index, edge_attr, Wrel1, brel1, Wroot1, Wrel2, brel2,
           Wroot2, Wrel3, brel3, Wroot3, Wf1, bf1, Wr1, br1, Wf2, bf2, Wr2,
           br2):
    src2 = edge_index[0].reshape(NW, RPW, B)
    dst2 = edge_index[1].reshape(NW, RPW, B)
    w2 = edge_attr.reshape(NW, RPW, B)
    z32 = jnp.zeros((SP, 32), F32)
    z64 = jnp.zeros((SP, 64), F32)

    W0 = jnp.concatenate(
        [Wrel1.T, Wf1[:, :128].T, Wf1[:, 128:256].T, Wroot1.T], axis=1)
    ts1, b1p, r1 = pl.pallas_call(
        _tc0_body,
        out_shape=[jax.ShapeDtypeStruct((N_NODES, 64), F32),
                   jax.ShapeDtypeStruct((N_NODES, 32), F32),
                   jax.ShapeDtypeStruct((N_NODES, 32), F32)],
    )(x, W0, bf1.reshape(1, 32))

    p1 = jnp.concatenate([Wf1[:, 256], Wr1[0], br1, jnp.zeros((15,), F32)])
    e1_2, agg1 = _sc_edge_pass(src2, dst2, w2, ts1, b1p, p1, z32)

    W1 = jnp.concatenate([Wf2[:, :32].T, Wf2[:, 32:64].T, Wroot2.T], axis=1)
    ts2, b2p, r2 = pl.pallas_call(
        _tc1_body,
        out_shape=[jax.ShapeDtypeStruct((N_NODES, 64), F32),
                   jax.ShapeDtypeStruct((N_NODES, 32), F32),
                   jax.ShapeDtypeStruct((N_NODES, 64), F32)],
    )(agg1, r1, brel1.reshape(1, 32), W1, bf2.reshape(1, 32))

    p2 = jnp.concatenate([Wf2[:, 64], Wr2[0], br2, jnp.zeros((15,), F32)])
    e2_2, agg2 = _sc_edge_pass(src2, dst2, e1_2, ts2, b2p, p2, z32)

    ts3, r3 = pl.pallas_call(
        _tc2_body,
        out_shape=[jax.ShapeDtypeStruct((N_NODES, 64), F32),
                   jax.ShapeDtypeStruct((N_NODES, 128), F32)],
    )(agg2, r2, Wrel2.T, brel2.reshape(1, 64), Wroot3.T)

    (agg3,) = _sc_agg_pass(src2, dst2, e2_2, ts3, z64)

    x3 = pl.pallas_call(
        _tc3_body,
        out_shape=jax.ShapeDtypeStruct((N_NODES, 128), F32),
    )(agg3, r3, Wrel3.T, brel3.reshape(1, 128))
    return x3


# speedup vs baseline: 1.2005x; 1.0266x over previous
import functools

import jax
import jax.numpy as jnp
from jax import lax
from jax.experimental import pallas as pl
from jax.experimental.pallas import tpu as pltpu
from jax.experimental.pallas import tpu_sc as plsc

N_NODES = 10000
N_EDGES = 320000
NC, NS, L = 2, 16, 16
NW = NC * NS
B = 80
RPW = N_EDGES // (NW * B)
NPAD = 10240
SP = NPAD // NS
F32 = jnp.float32

_MESH = plsc.VectorSubcoreMesh(core_axis_name="c", subcore_axis_name="s")
_SC_PARAMS = pltpu.CompilerParams(needs_layout_passes=False,
                                  use_tc_tiling_on_sc=False)


def _edge_pass_body(src_h, dst_h, w_h, ts_h, b_h, p_h, z_h, e_out, agg_out,
                    src_v, dst_v, w_v, e_v, ts_a, ts_b, b_a, b_b, msg_a,
                    msg_b, mtx, pbuf, agg_sh, s1a, s1b, s2a, s2b, sca, scb):
    cid = lax.axis_index("c")
    sid = lax.axis_index("s")
    wid = sid * NC + cid

    stg = (pltpu.make_async_copy(z_h, agg_sh.at[pl.ds(sid * SP, SP)], s1a),
           pltpu.make_async_copy(src_h.at[wid], src_v, s1b),
           pltpu.make_async_copy(dst_h.at[wid], dst_v, s2a),
           pltpu.make_async_copy(w_h.at[wid], w_v, s2b))
    for cp in stg:
        cp.start()
    pltpu.sync_copy(p_h, pbuf)
    for cp in stg:
        cp.wait()
    plsc.subcore_barrier()
    wc0 = pbuf[pl.ds(0, L)]
    wc1 = pbuf[pl.ds(L, L)]
    wr0 = pbuf[pl.ds(2 * L, L)]
    wr1 = pbuf[pl.ds(3 * L, L)]
    brv = pbuf[pl.ds(4 * L, L)]
    br = brv[0]
    iot = lax.iota(jnp.int32, L)

    def _g(bi, ts_buf, b_buf, sem_ts, sem_b):
        return (pltpu.make_async_copy(ts_h.at[src_v.at[bi]], ts_buf, sem_ts),
                pltpu.make_async_copy(b_h.at[dst_v.at[bi]], b_buf, sem_b))

    def _scd(bi, msg, sem):
        return pltpu.make_async_copy(msg, agg_sh.at[dst_v.at[bi]], sem)

    def _compute(bi, ts_rows, b_rows, msg):
        for g in range(B // L):
            wv = w_v[bi, pl.ds(g * L, L)]
            for j in range(L):
                e = g * L + j
                w = wv[j]
                u0 = ts_rows[e, pl.ds(2 * L, L)] + b_rows[e, pl.ds(0, L)] + w * wc0
                u1 = ts_rows[e, pl.ds(3 * L, L)] + b_rows[e, pl.ds(L, L)] + w * wc1
                mtx[j] = (jnp.maximum(u0, 0.0) * wr0
                          + jnp.maximum(u1, 0.0) * wr1)
                msg[e, pl.ds(0, L)] = w * ts_rows[e, pl.ds(0, L)]
                msg[e, pl.ds(L, L)] = w * ts_rows[e, pl.ds(L, L)]
            tot = jnp.full((L,), br, F32)
            for c in range(L):
                tot = tot + plsc.load_gather(
                    mtx, [iot, jnp.full((L,), c, jnp.int32)])
            e_v[bi, pl.ds(g * L, L)] = tot

    pltpu.make_async_copy(z_h.at[pl.ds(0, B)], msg_a, sca).start()
    pltpu.make_async_copy(z_h.at[pl.ds(0, B)], msg_b, scb).start()

    for cp in _g(0, ts_a, b_a, s1a, s2a):
        cp.start()

    def body(i, carry):
        bi0 = 2 * i
        bi1 = bi0 + 1
        for cp in _g(bi1, ts_b, b_b, s1b, s2b):
            cp.start()
        for cp in _g(bi0, ts_a, b_a, s1a, s2a):
            cp.wait()
        _scd(bi0, msg_a, sca).wait()
        _compute(bi0, ts_a, b_a, msg_a)
        _scd(bi0, msg_a, sca).start(add=True)
        for cp in _g(bi0 + 2, ts_a, b_a, s1a, s2a):
            cp.start()
        for cp in _g(bi1, ts_b, b_b, s1b, s2b):
            cp.wait()
        _scd(bi1, msg_b, scb).wait()
        _compute(bi1, ts_b, b_b, msg_b)
        _scd(bi1, msg_b, scb).start(add=True)
        return carry

    lax.fori_loop(0, (RPW - 1) // 2, body, 0)
    for cp in _g(RPW - 1, ts_a, b_a, s1a, s2a):
        cp.wait()
    _scd(RPW - 1, msg_a, sca).wait()
    _compute(RPW - 1, ts_a, b_a, msg_a)
    _scd(RPW - 1, msg_a, sca).start(add=True)

    _scd(RPW - 1, msg_a, sca).wait()
    _scd(RPW - 2, msg_b, scb).wait()

    ev_cp = pltpu.make_async_copy(e_v, e_out.at[wid], s1a)
    ev_cp.start()
    plsc.subcore_barrier()
    pltpu.sync_copy(agg_sh.at[pl.ds(sid * SP, SP)],
                    agg_out.at[cid, pl.ds(sid * SP, SP)])
    ev_cp.wait()


_sc_edge_pass = functools.partial(
    pl.kernel,
    out_type=[jax.ShapeDtypeStruct((NW, RPW, B), F32),
              jax.ShapeDtypeStruct((NC, NPAD, 2 * L), F32)],
    mesh=_MESH,
    compiler_params=_SC_PARAMS,
    scratch_types=[
        pltpu.VMEM((RPW, B), jnp.int32),
        pltpu.VMEM((RPW, B), jnp.int32),
        pltpu.VMEM((RPW, B), F32),
        pltpu.VMEM((RPW, B), F32),
        pltpu.VMEM((B, 4 * L), F32),
        pltpu.VMEM((B, 4 * L), F32),
        pltpu.VMEM((B, 2 * L), F32),
        pltpu.VMEM((B, 2 * L), F32),
        pltpu.VMEM((B, 2 * L), F32),
        pltpu.VMEM((B, 2 * L), F32),
        pltpu.VMEM((L, L), F32),
        pltpu.VMEM((5 * L,), F32),
        pltpu.VMEM_SHARED((NPAD, 2 * L), F32),
        pltpu.SemaphoreType.DMA,
        pltpu.SemaphoreType.DMA,
        pltpu.SemaphoreType.DMA,
        pltpu.SemaphoreType.DMA,
        pltpu.SemaphoreType.DMA,
        pltpu.SemaphoreType.DMA,
    ],
)(_edge_pass_body)


def _agg_pass_body(src_h, dst_h, w_h, ts_h, z_h, agg_out,
                   src_v, dst_v, w_v, ts_a, ts_b, msg_a, msg_b, agg_sh,
                   s1a, s1b, sca, scb):
    cid = lax.axis_index("c")
    sid = lax.axis_index("s")
    wid = sid * NC + cid

    stg = (pltpu.make_async_copy(z_h, agg_sh.at[pl.ds(sid * SP, SP)], s1a),
           pltpu.make_async_copy(src_h.at[wid], src_v, s1b),
           pltpu.make_async_copy(dst_h.at[wid], dst_v, sca),
           pltpu.make_async_copy(w_h.at[wid], w_v, scb))
    for cp in stg:
        cp.start()
    for cp in stg:
        cp.wait()
    plsc.subcore_barrier()

    def _g(bi, ts_buf, sem):
        return pltpu.make_async_copy(ts_h.at[src_v.at[bi]], ts_buf, sem)

    def _scd(bi, msg, sem):
        return pltpu.make_async_copy(msg, agg_sh.at[dst_v.at[bi]], sem)

    def _compute(bi, ts_rows, msg):
        for g in range(B // L):
            wv = w_v[bi, pl.ds(g * L, L)]
            for j in range(L):
                e = g * L + j
                w = wv[j]
                for k in range(4):
                    msg[e, pl.ds(k * L, L)] = w * ts_rows[e, pl.ds(k * L, L)]

    pltpu.make_async_copy(z_h.at[pl.ds(0, B)], msg_a, sca).start()
    pltpu.make_async_copy(z_h.at[pl.ds(0, B)], msg_b, scb).start()

    _g(0, ts_a, s1a).start()

    def body(i, carry):
        bi0 = 2 * i
        bi1 = bi0 + 1
        _g(bi1, ts_b, s1b).start()
        _g(bi0, ts_a, s1a).wait()
        _scd(bi0, msg_a, sca).wait()
        _compute(bi0, ts_a, msg_a)
        _scd(bi0, msg_a, sca).start(add=True)
        _g(bi0 + 2, ts_a, s1a).start()
        _g(bi1, ts_b, s1b).wait()
        _scd(bi1, msg_b, scb).wait()
        _compute(bi1, ts_b, msg_b)
        _scd(bi1, msg_b, scb).start(add=True)
        return carry

    lax.fori_loop(0, (RPW - 1) // 2, body, 0)
    _g(RPW - 1, ts_a, s1a).wait()
    _scd(RPW - 1, msg_a, sca).wait()
    _compute(RPW - 1, ts_a, msg_a)
    _scd(RPW - 1, msg_a, sca).start(add=True)

    _scd(RPW - 1, msg_a, sca).wait()
    _scd(RPW - 2, msg_b, scb).wait()

    plsc.subcore_barrier()
    pltpu.sync_copy(agg_sh.at[pl.ds(sid * SP, SP)],
                    agg_out.at[cid, pl.ds(sid * SP, SP)])


_sc_agg_pass = functools.partial(
    pl.kernel,
    out_type=[jax.ShapeDtypeStruct((NC, NPAD, 4 * L), F32)],
    mesh=_MESH,
    compiler_params=_SC_PARAMS,
    scratch_types=[
        pltpu.VMEM((RPW, B), jnp.int32),
        pltpu.VMEM((RPW, B), jnp.int32),
        pltpu.VMEM((RPW, B), F32),
        pltpu.VMEM((B, 4 * L), F32),
        pltpu.VMEM((B, 4 * L), F32),
        pltpu.VMEM((B, 4 * L), F32),
        pltpu.VMEM((B, 4 * L), F32),
        pltpu.VMEM_SHARED((NPAD, 4 * L), F32),
        pltpu.SemaphoreType.DMA,
        pltpu.SemaphoreType.DMA,
        pltpu.SemaphoreType.DMA,
        pltpu.SemaphoreType.DMA,
    ],
)(_agg_pass_body)


def _tc0_body(x_ref, w_ref, bf_ref, ts_ref, b_ref, r_ref):
    y = jnp.dot(x_ref[...], w_ref[...], preferred_element_type=F32)
    ts_ref[...] = y[:, 0:64]
    b_ref[...] = y[:, 64:96] + bf_ref[...]
    r_ref[...] = y[:, 96:128]


def _tc1_body(agg_ref, r_ref, brel_ref, w_ref, bf_ref, ts_ref, b_ref, r2_ref):
    agg = agg_ref[0, :N_NODES] + agg_ref[1, :N_NODES]
    x1 = jnp.maximum(agg + brel_ref[...] + r_ref[...], 0.0)
    y = jnp.dot(x1, w_ref[...], preferred_element_type=F32)
    ts_ref[...] = jnp.concatenate([x1, y[:, 0:32]], axis=1)
    b_ref[...] = y[:, 32:64] + bf_ref[...]
    r2_ref[...] = y[:, 64:128]


def _tc2_body(agg_ref, r_ref, wrel_ref, brel_ref, wroot_ref, ts_ref, r3_ref):
    a = agg_ref[0, :N_NODES] + agg_ref[1, :N_NODES]
    x2 = jnp.maximum(jnp.dot(a, wrel_ref[...], preferred_element_type=F32)
                     + brel_ref[...] + r_ref[...], 0.0)
    ts_ref[...] = x2
    r3_ref[...] = jnp.dot(x2, wroot_ref[...], preferred_element_type=F32)


def _tc3_body(agg_ref, r_ref, wrel_ref, brel_ref, out_ref):
    a = agg_ref[0, :N_NODES] + agg_ref[1, :N_NODES]
    out_ref[...] = jnp.maximum(
        jnp.dot(a, wrel_ref[...], preferred_element_type=F32)
        + brel_ref[...] + r_ref[...], 0.0)


def kernel(x, edge_index, edge_attr, Wrel1, brel1, Wroot1, Wrel2, brel2,
           Wroot2, Wrel3, brel3, Wroot3, Wf1, bf1, Wr1, br1, Wf2, bf2, Wr2,
           br2):
    src2 = edge_index[0].reshape(NW, RPW, B)
    dst2 = edge_index[1].reshape(NW, RPW, B)
    w2 = edge_attr.reshape(NW, RPW, B)
    z32 = jnp.zeros((SP, 32), F32)
    z64 = jnp.zeros((SP, 64), F32)

    W0 = jnp.concatenate(
        [Wrel1.T, Wf1[:, :128].T, Wf1[:, 128:256].T, Wroot1.T], axis=1)
    ts1, b1p, r1 = pl.pallas_call(
        _tc0_body,
        out_shape=[jax.ShapeDtypeStruct((N_NODES, 64), F32),
                   jax.ShapeDtypeStruct((N_NODES, 32), F32),
                   jax.ShapeDtypeStruct((N_NODES, 32), F32)],
    )(x, W0, bf1.reshape(1, 32))

    p1 = jnp.concatenate([Wf1[:, 256], Wr1[0], br1, jnp.zeros((15,), F32)])
    e1_2, agg1 = _sc_edge_pass(src2, dst2, w2, ts1, b1p, p1, z32)

    W1 = jnp.concatenate([Wf2[:, :32].T, Wf2[:, 32:64].T, Wroot2.T], axis=1)
    ts2, b2p, r2 = pl.pallas_call(
        _tc1_body,
        out_shape=[jax.ShapeDtypeStruct((N_NODES, 64), F32),
                   jax.ShapeDtypeStruct((N_NODES, 32), F32),
                   jax.ShapeDtypeStruct((N_NODES, 64), F32)],
    )(agg1, r1, brel1.reshape(1, 32), W1, bf2.reshape(1, 32))

    p2 = jnp.concatenate([Wf2[:, 64], Wr2[0], br2, jnp.zeros((15,), F32)])
    e2_2, agg2 = _sc_edge_pass(src2, dst2, e1_2, ts2, b2p, p2, z32)

    ts3, r3 = pl.pallas_call(
        _tc2_body,
        out_shape=[jax.ShapeDtypeStruct((N_NODES, 64), F32),
                   jax.ShapeDtypeStruct((N_NODES, 128), F32)],
    )(agg2, r2, Wrel2.T, brel2.reshape(1, 64), Wroot3.T)

    (agg3,) = _sc_agg_pass(src2, dst2, e2_2, ts3, z64)

    x3 = pl.pallas_call(
        _tc3_body,
        out_shape=jax.ShapeDtypeStruct((N_NODES, 128), F32),
    )(agg3, r3, Wrel3.T, brel3.reshape(1, 128))
    return x3


# speedup vs baseline: 1.2777x; 1.0643x over previous
import functools

import jax
import jax.numpy as jnp
from jax import lax
from jax.experimental import pallas as pl
from jax.experimental.pallas import tpu as pltpu
from jax.experimental.pallas import tpu_sc as plsc

N_NODES = 10000
N_EDGES = 320000
NC, NS, L = 2, 16, 16
NW = NC * NS
B = 80
RPW = N_EDGES // (NW * B)
NPAD = 10240
SP = NPAD // NS
F32 = jnp.float32

_MESH = plsc.VectorSubcoreMesh(core_axis_name="c", subcore_axis_name="s")
_SC_PARAMS = pltpu.CompilerParams(needs_layout_passes=False,
                                  use_tc_tiling_on_sc=False)


def _edge_pass_body(src_h, dst_h, w_h, ts_h, b_h, p_h, z_h, e_out, agg_out,
                    src_v, dst_v, w_v, e_v, ts_a, ts_b, b_a, b_b, msg_a,
                    msg_b, pbuf, agg_sh, s1a, s1b, s2a, s2b, sca, scb):
    cid = lax.axis_index("c")
    sid = lax.axis_index("s")
    wid = sid * NC + cid

    stg = (pltpu.make_async_copy(z_h, agg_sh.at[pl.ds(sid * SP, SP)], s1a),
           pltpu.make_async_copy(src_h.at[wid], src_v, s1b),
           pltpu.make_async_copy(dst_h.at[wid], dst_v, s2a),
           pltpu.make_async_copy(w_h.at[wid], w_v, s2b))
    for cp in stg:
        cp.start()
    pltpu.sync_copy(p_h, pbuf)
    for cp in stg:
        cp.wait()
    plsc.subcore_barrier()
    wc0 = pbuf[pl.ds(0, L)]
    wc1 = pbuf[pl.ds(L, L)]
    wr0 = pbuf[pl.ds(2 * L, L)]
    wr1 = pbuf[pl.ds(3 * L, L)]
    brv = pbuf[pl.ds(4 * L, L)]
    br = brv[0]
    iot = lax.iota(jnp.int32, L)

    def _g(bi, ts_buf, b_buf, sem_ts, sem_b):
        return (pltpu.make_async_copy(ts_h.at[src_v.at[bi]], ts_buf, sem_ts),
                pltpu.make_async_copy(b_h.at[dst_v.at[bi]], b_buf, sem_b))

    def _scd(bi, msg, sem):
        return pltpu.make_async_copy(msg, agg_sh.at[dst_v.at[bi]], sem)

    def _compute(bi, ts_rows, b_rows, msg):
        for g in range(B // L):
            wv = w_v[bi, pl.ds(g * L, L)]
            acc = jnp.zeros((L,), F32)
            for j in range(L):
                e = g * L + j
                w = wv[j]
                u0 = ts_rows[e, pl.ds(2 * L, L)] + b_rows[e, pl.ds(0, L)] + w * wc0
                u1 = ts_rows[e, pl.ds(3 * L, L)] + b_rows[e, pl.ds(L, L)] + w * wc1
                t = (jnp.maximum(u0, 0.0) * wr0
                     + jnp.maximum(u1, 0.0) * wr1)
                cs = plsc.cumsum(t)
                acc = jnp.where(iot == j, cs[L - 1], acc)
                msg[e, pl.ds(0, L)] = w * ts_rows[e, pl.ds(0, L)]
                msg[e, pl.ds(L, L)] = w * ts_rows[e, pl.ds(L, L)]
            e_v[bi, pl.ds(g * L, L)] = acc + br

    pltpu.make_async_copy(z_h.at[pl.ds(0, B)], msg_a, sca).start()
    pltpu.make_async_copy(z_h.at[pl.ds(0, B)], msg_b, scb).start()

    for cp in _g(0, ts_a, b_a, s1a, s2a):
        cp.start()

    def body(i, carry):
        bi0 = 2 * i
        bi1 = bi0 + 1
        for cp in _g(bi1, ts_b, b_b, s1b, s2b):
            cp.start()
        for cp in _g(bi0, ts_a, b_a, s1a, s2a):
            cp.wait()
        _scd(bi0, msg_a, sca).wait()
        _compute(bi0, ts_a, b_a, msg_a)
        _scd(bi0, msg_a, sca).start(add=True)
        for cp in _g(bi0 + 2, ts_a, b_a, s1a, s2a):
            cp.start()
        for cp in _g(bi1, ts_b, b_b, s1b, s2b):
            cp.wait()
        _scd(bi1, msg_b, scb).wait()
        _compute(bi1, ts_b, b_b, msg_b)
        _scd(bi1, msg_b, scb).start(add=True)
        return carry

    lax.fori_loop(0, (RPW - 1) // 2, body, 0)
    for cp in _g(RPW - 1, ts_a, b_a, s1a, s2a):
        cp.wait()
    _scd(RPW - 1, msg_a, sca).wait()
    _compute(RPW - 1, ts_a, b_a, msg_a)
    _scd(RPW - 1, msg_a, sca).start(add=True)

    _scd(RPW - 1, msg_a, sca).wait()
    _scd(RPW - 2, msg_b, scb).wait()

    ev_cp = pltpu.make_async_copy(e_v, e_out.at[wid], s1a)
    ev_cp.start()
    plsc.subcore_barrier()
    pltpu.sync_copy(agg_sh.at[pl.ds(sid * SP, SP)],
                    agg_out.at[cid, pl.ds(sid * SP, SP)])
    ev_cp.wait()


_sc_edge_pass = functools.partial(
    pl.kernel,
    out_type=[jax.ShapeDtypeStruct((NW, RPW, B), F32),
              jax.ShapeDtypeStruct((NC, NPAD, 2 * L), F32)],
    mesh=_MESH,
    compiler_params=_SC_PARAMS,
    scratch_types=[
        pltpu.VMEM((RPW, B), jnp.int32),
        pltpu.VMEM((RPW, B), jnp.int32),
        pltpu.VMEM((RPW, B), F32),
        pltpu.VMEM((RPW, B), F32),
        pltpu.VMEM((B, 4 * L), F32),
        pltpu.VMEM((B, 4 * L), F32),
        pltpu.VMEM((B, 2 * L), F32),
        pltpu.VMEM((B, 2 * L), F32),
        pltpu.VMEM((B, 2 * L), F32),
        pltpu.VMEM((B, 2 * L), F32),
        pltpu.VMEM((5 * L,), F32),
        pltpu.VMEM_SHARED((NPAD, 2 * L), F32),
        pltpu.SemaphoreType.DMA,
        pltpu.SemaphoreType.DMA,
        pltpu.SemaphoreType.DMA,
        pltpu.SemaphoreType.DMA,
        pltpu.SemaphoreType.DMA,
        pltpu.SemaphoreType.DMA,
    ],
)(_edge_pass_body)


def _agg_pass_body(src_h, dst_h, w_h, ts_h, z_h, agg_out,
                   src_v, dst_v, w_v, ts_a, ts_b, msg_a, msg_b, agg_sh,
                   s1a, s1b, sca, scb):
    cid = lax.axis_index("c")
    sid = lax.axis_index("s")
    wid = sid * NC + cid

    stg = (pltpu.make_async_copy(z_h, agg_sh.at[pl.ds(sid * SP, SP)], s1a),
           pltpu.make_async_copy(src_h.at[wid], src_v, s1b),
           pltpu.make_async_copy(dst_h.at[wid], dst_v, sca),
           pltpu.make_async_copy(w_h.at[wid], w_v, scb))
    for cp in stg:
        cp.start()
    for cp in stg:
        cp.wait()
    plsc.subcore_barrier()

    def _g(bi, ts_buf, sem):
        return pltpu.make_async_copy(ts_h.at[src_v.at[bi]], ts_buf, sem)

    def _scd(bi, msg, sem):
        return pltpu.make_async_copy(msg, agg_sh.at[dst_v.at[bi]], sem)

    def _compute(bi, ts_rows, msg):
        for g in range(B // L):
            wv = w_v[bi, pl.ds(g * L, L)]
            for j in range(L):
                e = g * L + j
                w = wv[j]
                for k in range(4):
                    msg[e, pl.ds(k * L, L)] = w * ts_rows[e, pl.ds(k * L, L)]

    pltpu.make_async_copy(z_h.at[pl.ds(0, B)], msg_a, sca).start()
    pltpu.make_async_copy(z_h.at[pl.ds(0, B)], msg_b, scb).start()

    _g(0, ts_a, s1a).start()

    def body(i, carry):
        bi0 = 2 * i
        bi1 = bi0 + 1
        _g(bi1, ts_b, s1b).start()
        _g(bi0, ts_a, s1a).wait()
        _scd(bi0, msg_a, sca).wait()
        _compute(bi0, ts_a, msg_a)
        _scd(bi0, msg_a, sca).start(add=True)
        _g(bi0 + 2, ts_a, s1a).start()
        _g(bi1, ts_b, s1b).wait()
        _scd(bi1, msg_b, scb).wait()
        _compute(bi1, ts_b, msg_b)
        _scd(bi1, msg_b, scb).start(add=True)
        return carry

    lax.fori_loop(0, (RPW - 1) // 2, body, 0)
    _g(RPW - 1, ts_a, s1a).wait()
    _scd(RPW - 1, msg_a, sca).wait()
    _compute(RPW - 1, ts_a, msg_a)
    _scd(RPW - 1, msg_a, sca).start(add=True)

    _scd(RPW - 1, msg_a, sca).wait()
    _scd(RPW - 2, msg_b, scb).wait()

    plsc.subcore_barrier()
    pltpu.sync_copy(agg_sh.at[pl.ds(sid * SP, SP)],
                    agg_out.at[cid, pl.ds(sid * SP, SP)])


_sc_agg_pass = functools.partial(
    pl.kernel,
    out_type=[jax.ShapeDtypeStruct((NC, NPAD, 4 * L), F32)],
    mesh=_MESH,
    compiler_params=_SC_PARAMS,
    scratch_types=[
        pltpu.VMEM((RPW, B), jnp.int32),
        pltpu.VMEM((RPW, B), jnp.int32),
        pltpu.VMEM((RPW, B), F32),
        pltpu.VMEM((B, 4 * L), F32),
        pltpu.VMEM((B, 4 * L), F32),
        pltpu.VMEM((B, 4 * L), F32),
        pltpu.VMEM((B, 4 * L), F32),
        pltpu.VMEM_SHARED((NPAD, 4 * L), F32),
        pltpu.SemaphoreType.DMA,
        pltpu.SemaphoreType.DMA,
        pltpu.SemaphoreType.DMA,
        pltpu.SemaphoreType.DMA,
    ],
)(_agg_pass_body)


def _tc0_body(x_ref, w_ref, bf_ref, ts_ref, b_ref, r_ref):
    y = jnp.dot(x_ref[...], w_ref[...], preferred_element_type=F32)
    ts_ref[...] = y[:, 0:64]
    b_ref[...] = y[:, 64:96] + bf_ref[...]
    r_ref[...] = y[:, 96:128]


def _tc1_body(agg_ref, r_ref, brel_ref, w_ref, bf_ref, ts_ref, b_ref, r2_ref):
    agg = agg_ref[0, :N_NODES] + agg_ref[1, :N_NODES]
    x1 = jnp.maximum(agg + brel_ref[...] + r_ref[...], 0.0)
    y = jnp.dot(x1, w_ref[...], preferred_element_type=F32)
    ts_ref[...] = jnp.concatenate([x1, y[:, 0:32]], axis=1)
    b_ref[...] = y[:, 32:64] + bf_ref[...]
    r2_ref[...] = y[:, 64:128]


def _tc2_body(agg_ref, r_ref, wrel_ref, brel_ref, wroot_ref, ts_ref, r3_ref):
    a = agg_ref[0, :N_NODES] + agg_ref[1, :N_NODES]
    x2 = jnp.maximum(jnp.dot(a, wrel_ref[...], preferred_element_type=F32)
                     + brel_ref[...] + r_ref[...], 0.0)
    ts_ref[...] = x2
    r3_ref[...] = jnp.dot(x2, wroot_ref[...], preferred_element_type=F32)


def _tc3_body(agg_ref, r_ref, wrel_ref, brel_ref, out_ref):
    a = agg_ref[0, :N_NODES] + agg_ref[1, :N_NODES]
    out_ref[...] = jnp.maximum(
        jnp.dot(a, wrel_ref[...], preferred_element_type=F32)
        + brel_ref[...] + r_ref[...], 0.0)


def kernel(x, edge_index, edge_attr, Wrel1, brel1, Wroot1, Wrel2, brel2,
           Wroot2, Wrel3, brel3, Wroot3, Wf1, bf1, Wr1, br1, Wf2, bf2, Wr2,
           br2):
    src2 = edge_index[0].reshape(NW, RPW, B)
    dst2 = edge_index[1].reshape(NW, RPW, B)
    w2 = edge_attr.reshape(NW, RPW, B)
    z32 = jnp.zeros((SP, 32), F32)
    z64 = jnp.zeros((SP, 64), F32)

    W0 = jnp.concatenate(
        [Wrel1.T, Wf1[:, :128].T, Wf1[:, 128:256].T, Wroot1.T], axis=1)
    ts1, b1p, r1 = pl.pallas_call(
        _tc0_body,
        out_shape=[jax.ShapeDtypeStruct((N_NODES, 64), F32),
                   jax.ShapeDtypeStruct((N_NODES, 32), F32),
                   jax.ShapeDtypeStruct((N_NODES, 32), F32)],
    )(x, W0, bf1.reshape(1, 32))

    p1 = jnp.concatenate([Wf1[:, 256], Wr1[0], br1, jnp.zeros((15,), F32)])
    e1_2, agg1 = _sc_edge_pass(src2, dst2, w2, ts1, b1p, p1, z32)

    W1 = jnp.concatenate([Wf2[:, :32].T, Wf2[:, 32:64].T, Wroot2.T], axis=1)
    ts2, b2p, r2 = pl.pallas_call(
        _tc1_body,
        out_shape=[jax.ShapeDtypeStruct((N_NODES, 64), F32),
                   jax.ShapeDtypeStruct((N_NODES, 32), F32),
                   jax.ShapeDtypeStruct((N_NODES, 64), F32)],
    )(agg1, r1, brel1.reshape(1, 32), W1, bf2.reshape(1, 32))

    p2 = jnp.concatenate([Wf2[:, 64], Wr2[0], br2, jnp.zeros((15,), F32)])
    e2_2, agg2 = _sc_edge_pass(src2, dst2, e1_2, ts2, b2p, p2, z32)

    ts3, r3 = pl.pallas_call(
        _tc2_body,
        out_shape=[jax.ShapeDtypeStruct((N_NODES, 64), F32),
                   jax.ShapeDtypeStruct((N_NODES, 128), F32)],
    )(agg2, r2, Wrel2.T, brel2.reshape(1, 64), Wroot3.T)

    (agg3,) = _sc_agg_pass(src2, dst2, e2_2, ts3, z64)

    x3 = pl.pallas_call(
        _tc3_body,
        out_shape=jax.ShapeDtypeStruct((N_NODES, 128), F32),
    )(agg3, r3, Wrel3.T, brel3.reshape(1, 128))
    return x3


# speedup vs baseline: 1.3110x; 1.0260x over previous
import functools

import jax
import jax.numpy as jnp
from jax import lax
from jax.experimental import pallas as pl
from jax.experimental.pallas import tpu as pltpu
from jax.experimental.pallas import tpu_sc as plsc

N_NODES = 10000
N_EDGES = 320000
NC, NS, L = 2, 16, 16
NW = NC * NS
B = 80
RPW = N_EDGES // (NW * B)
NPAD = 10240
SP = NPAD // NS
F32 = jnp.float32

_MESH = plsc.VectorSubcoreMesh(core_axis_name="c", subcore_axis_name="s")
_SC_PARAMS = pltpu.CompilerParams(needs_layout_passes=False,
                                  use_tc_tiling_on_sc=False)


def _edge_pass_body(ei_h, w_h, ts_h, b_h, p_h, z_h, e_out, agg_out,
                    src_v, dst_v, w_v, e_v, ts_a, ts_b, b_a, b_b, msg_a,
                    msg_b, pbuf, agg_sh, s1a, s1b, s2a, s2b, sca, scb):
    cid = lax.axis_index("c")
    sid = lax.axis_index("s")
    wid = sid * NC + cid

    stg = (pltpu.make_async_copy(z_h, agg_sh.at[pl.ds(sid * SP, SP)], s1a),
           pltpu.make_async_copy(ei_h.at[0, wid], src_v, s1b),
           pltpu.make_async_copy(ei_h.at[1, wid], dst_v, s2a),
           pltpu.make_async_copy(w_h.at[wid], w_v, s2b))
    for cp in stg:
        cp.start()
    pltpu.sync_copy(p_h, pbuf)
    for cp in stg:
        cp.wait()
    plsc.subcore_barrier()
    wc0 = pbuf[pl.ds(0, L)]
    wc1 = pbuf[pl.ds(L, L)]
    wr0 = pbuf[pl.ds(2 * L, L)]
    wr1 = pbuf[pl.ds(3 * L, L)]
    brv = pbuf[pl.ds(4 * L, L)]
    br = brv[0]
    iot = lax.iota(jnp.int32, L)

    def _g(bi, ts_buf, b_buf, sem_ts, sem_b):
        return (pltpu.make_async_copy(ts_h.at[src_v.at[bi]], ts_buf, sem_ts),
                pltpu.make_async_copy(b_h.at[dst_v.at[bi]], b_buf, sem_b))

    def _scd(bi, msg, sem):
        return pltpu.make_async_copy(msg, agg_sh.at[dst_v.at[bi]], sem)

    def _compute(bi, ts_rows, b_rows, msg):
        for g in range(B // L):
            wv = w_v[bi, pl.ds(g * L, L)]
            acc = jnp.zeros((L,), F32)
            for j in range(L):
                e = g * L + j
                w = wv[j]
                u0 = ts_rows[e, pl.ds(2 * L, L)] + b_rows[e, pl.ds(0, L)] + w * wc0
                u1 = ts_rows[e, pl.ds(3 * L, L)] + b_rows[e, pl.ds(L, L)] + w * wc1
                t = (jnp.maximum(u0, 0.0) * wr0
                     + jnp.maximum(u1, 0.0) * wr1)
                cs = plsc.cumsum(t)
                acc = jnp.where(iot == j, cs[L - 1], acc)
                msg[e, pl.ds(0, L)] = w * ts_rows[e, pl.ds(0, L)]
                msg[e, pl.ds(L, L)] = w * ts_rows[e, pl.ds(L, L)]
            e_v[bi, pl.ds(g * L, L)] = acc + br

    pltpu.make_async_copy(z_h.at[pl.ds(0, B)], msg_a, sca).start()
    pltpu.make_async_copy(z_h.at[pl.ds(0, B)], msg_b, scb).start()

    for cp in _g(0, ts_a, b_a, s1a, s2a):
        cp.start()

    def body(i, carry):
        bi0 = 2 * i
        bi1 = bi0 + 1
        for cp in _g(bi1, ts_b, b_b, s1b, s2b):
            cp.start()
        for cp in _g(bi0, ts_a, b_a, s1a, s2a):
            cp.wait()
        _scd(bi0, msg_a, sca).wait()
        _compute(bi0, ts_a, b_a, msg_a)
        _scd(bi0, msg_a, sca).start(add=True)
        for cp in _g(bi0 + 2, ts_a, b_a, s1a, s2a):
            cp.start()
        for cp in _g(bi1, ts_b, b_b, s1b, s2b):
            cp.wait()
        _scd(bi1, msg_b, scb).wait()
        _compute(bi1, ts_b, b_b, msg_b)
        _scd(bi1, msg_b, scb).start(add=True)
        return carry

    lax.fori_loop(0, (RPW - 1) // 2, body, 0)
    for cp in _g(RPW - 1, ts_a, b_a, s1a, s2a):
        cp.wait()
    _scd(RPW - 1, msg_a, sca).wait()
    _compute(RPW - 1, ts_a, b_a, msg_a)
    _scd(RPW - 1, msg_a, sca).start(add=True)

    _scd(RPW - 1, msg_a, sca).wait()
    _scd(RPW - 2, msg_b, scb).wait()

    ev_cp = pltpu.make_async_copy(e_v, e_out.at[wid], s1a)
    ev_cp.start()
    plsc.subcore_barrier()
    pltpu.sync_copy(agg_sh.at[pl.ds(sid * SP, SP)],
                    agg_out.at[cid, pl.ds(sid * SP, SP)])
    ev_cp.wait()


_sc_edge_pass = functools.partial(
    pl.kernel,
    out_type=[jax.ShapeDtypeStruct((NW, RPW, B), F32),
              jax.ShapeDtypeStruct((NC, NPAD, 2 * L), F32)],
    mesh=_MESH,
    compiler_params=_SC_PARAMS,
    scratch_types=[
        pltpu.VMEM((RPW, B), jnp.int32),
        pltpu.VMEM((RPW, B), jnp.int32),
        pltpu.VMEM((RPW, B), F32),
        pltpu.VMEM((RPW, B), F32),
        pltpu.VMEM((B, 4 * L), F32),
        pltpu.VMEM((B, 4 * L), F32),
        pltpu.VMEM((B, 2 * L), F32),
        pltpu.VMEM((B, 2 * L), F32),
        pltpu.VMEM((B, 2 * L), F32),
        pltpu.VMEM((B, 2 * L), F32),
        pltpu.VMEM((5 * L,), F32),
        pltpu.VMEM_SHARED((NPAD, 2 * L), F32),
        pltpu.SemaphoreType.DMA,
        pltpu.SemaphoreType.DMA,
        pltpu.SemaphoreType.DMA,
        pltpu.SemaphoreType.DMA,
        pltpu.SemaphoreType.DMA,
        pltpu.SemaphoreType.DMA,
    ],
)(_edge_pass_body)


def _agg_pass_body(ei_h, w_h, ts_h, z_h, agg_out,
                   src_v, dst_v, w_v, ts_a, ts_b, msg_a, msg_b, agg_sh,
                   s1a, s1b, sca, scb):
    cid = lax.axis_index("c")
    sid = lax.axis_index("s")
    wid = sid * NC + cid

    stg = (pltpu.make_async_copy(z_h, agg_sh.at[pl.ds(sid * SP, SP)], s1a),
           pltpu.make_async_copy(ei_h.at[0, wid], src_v, s1b),
           pltpu.make_async_copy(ei_h.at[1, wid], dst_v, sca),
           pltpu.make_async_copy(w_h.at[wid], w_v, scb))
    for cp in stg:
        cp.start()
    for cp in stg:
        cp.wait()
    plsc.subcore_barrier()

    def _g(bi, ts_buf, sem):
        return pltpu.make_async_copy(ts_h.at[src_v.at[bi]], ts_buf, sem)

    def _scd(bi, msg, sem):
        return pltpu.make_async_copy(msg, agg_sh.at[dst_v.at[bi]], sem)

    def _compute(bi, ts_rows, msg):
        for g in range(B // L):
            wv = w_v[bi, pl.ds(g * L, L)]
            for j in range(L):
                e = g * L + j
                w = wv[j]
                for k in range(4):
                    msg[e, pl.ds(k * L, L)] = w * ts_rows[e, pl.ds(k * L, L)]

    pltpu.make_async_copy(z_h.at[pl.ds(0, B)], msg_a, sca).start()
    pltpu.make_async_copy(z_h.at[pl.ds(0, B)], msg_b, scb).start()

    _g(0, ts_a, s1a).start()

    def body(i, carry):
        bi0 = 2 * i
        bi1 = bi0 + 1
        _g(bi1, ts_b, s1b).start()
        _g(bi0, ts_a, s1a).wait()
        _scd(bi0, msg_a, sca).wait()
        _compute(bi0, ts_a, msg_a)
        _scd(bi0, msg_a, sca).start(add=True)
        _g(bi0 + 2, ts_a, s1a).start()
        _g(bi1, ts_b, s1b).wait()
        _scd(bi1, msg_b, scb).wait()
        _compute(bi1, ts_b, msg_b)
        _scd(bi1, msg_b, scb).start(add=True)
        return carry

    lax.fori_loop(0, (RPW - 1) // 2, body, 0)
    _g(RPW - 1, ts_a, s1a).wait()
    _scd(RPW - 1, msg_a, sca).wait()
    _compute(RPW - 1, ts_a, msg_a)
    _scd(RPW - 1, msg_a, sca).start(add=True)

    _scd(RPW - 1, msg_a, sca).wait()
    _scd(RPW - 2, msg_b, scb).wait()

    plsc.subcore_barrier()
    pltpu.sync_copy(agg_sh.at[pl.ds(sid * SP, SP)],
                    agg_out.at[cid, pl.ds(sid * SP, SP)])


_sc_agg_pass = functools.partial(
    pl.kernel,
    out_type=[jax.ShapeDtypeStruct((NC, NPAD, 4 * L), F32)],
    mesh=_MESH,
    compiler_params=_SC_PARAMS,
    scratch_types=[
        pltpu.VMEM((RPW, B), jnp.int32),
        pltpu.VMEM((RPW, B), jnp.int32),
        pltpu.VMEM((RPW, B), F32),
        pltpu.VMEM((B, 4 * L), F32),
        pltpu.VMEM((B, 4 * L), F32),
        pltpu.VMEM((B, 4 * L), F32),
        pltpu.VMEM((B, 4 * L), F32),
        pltpu.VMEM_SHARED((NPAD, 4 * L), F32),
        pltpu.SemaphoreType.DMA,
        pltpu.SemaphoreType.DMA,
        pltpu.SemaphoreType.DMA,
        pltpu.SemaphoreType.DMA,
    ],
)(_agg_pass_body)


def _tc0_body(x_ref, w_ref, bf_ref, ts_ref, b_ref, r_ref):
    y = jnp.dot(x_ref[...], w_ref[...], preferred_element_type=F32)
    ts_ref[...] = y[:, 0:64]
    b_ref[...] = y[:, 64:96] + bf_ref[...]
    r_ref[...] = y[:, 96:128]


def _tc1_body(agg_ref, r_ref, brel_ref, w_ref, bf_ref, ts_ref, b_ref, r2_ref):
    agg = agg_ref[0, :N_NODES] + agg_ref[1, :N_NODES]
    x1 = jnp.maximum(agg + brel_ref[...] + r_ref[...], 0.0)
    y = jnp.dot(x1, w_ref[...], preferred_element_type=F32)
    ts_ref[...] = jnp.concatenate([x1, y[:, 0:32]], axis=1)
    b_ref[...] = y[:, 32:64] + bf_ref[...]
    r2_ref[...] = y[:, 64:128]


def _tc2_body(agg_ref, r_ref, wrel_ref, brel_ref, wroot_ref, ts_ref, r3_ref):
    a = agg_ref[0, :N_NODES] + agg_ref[1, :N_NODES]
    x2 = jnp.maximum(jnp.dot(a, wrel_ref[...], preferred_element_type=F32)
                     + brel_ref[...] + r_ref[...], 0.0)
    ts_ref[...] = x2
    r3_ref[...] = jnp.dot(x2, wroot_ref[...], preferred_element_type=F32)


def _tc3_body(agg_ref, r_ref, wrel_ref, brel_ref, out_ref):
    a = agg_ref[0, :N_NODES] + agg_ref[1, :N_NODES]
    out_ref[...] = jnp.maximum(
        jnp.dot(a, wrel_ref[...], preferred_element_type=F32)
        + brel_ref[...] + r_ref[...], 0.0)


def kernel(x, edge_index, edge_attr, Wrel1, brel1, Wroot1, Wrel2, brel2,
           Wroot2, Wrel3, brel3, Wroot3, Wf1, bf1, Wr1, br1, Wf2, bf2, Wr2,
           br2):
    ei3 = edge_index.reshape(2, NW, RPW, B)
    w2 = edge_attr.reshape(NW, RPW, B)
    z32 = jnp.zeros((SP, 32), F32)
    z64 = jnp.zeros((SP, 64), F32)

    W0 = jnp.concatenate(
        [Wrel1.T, Wf1[:, :128].T, Wf1[:, 128:256].T, Wroot1.T], axis=1)
    ts1, b1p, r1 = pl.pallas_call(
        _tc0_body,
        out_shape=[jax.ShapeDtypeStruct((N_NODES, 64), F32),
                   jax.ShapeDtypeStruct((N_NODES, 32), F32),
                   jax.ShapeDtypeStruct((N_NODES, 32), F32)],
    )(x, W0, bf1.reshape(1, 32))

    p1 = jnp.concatenate([Wf1[:, 256], Wr1[0], br1, jnp.zeros((15,), F32)])
    e1_2, agg1 = _sc_edge_pass(ei3, w2, ts1, b1p, p1, z32)

    W1 = jnp.concatenate([Wf2[:, :32].T, Wf2[:, 32:64].T, Wroot2.T], axis=1)
    ts2, b2p, r2 = pl.pallas_call(
        _tc1_body,
        out_shape=[jax.ShapeDtypeStruct((N_NODES, 64), F32),
                   jax.ShapeDtypeStruct((N_NODES, 32), F32),
                   jax.ShapeDtypeStruct((N_NODES, 64), F32)],
    )(agg1, r1, brel1.reshape(1, 32), W1, bf2.reshape(1, 32))

    p2 = jnp.concatenate([Wf2[:, 64], Wr2[0], br2, jnp.zeros((15,), F32)])
    e2_2, agg2 = _sc_edge_pass(ei3, e1_2, ts2, b2p, p2, z32)

    ts3, r3 = pl.pallas_call(
        _tc2_body,
        out_shape=[jax.ShapeDtypeStruct((N_NODES, 64), F32),
                   jax.ShapeDtypeStruct((N_NODES, 128), F32)],
    )(agg2, r2, Wrel2.T, brel2.reshape(1, 64), Wroot3.T)

    (agg3,) = _sc_agg_pass(ei3, e2_2, ts3, z64)

    x3 = pl.pallas_call(
        _tc3_body,
        out_shape=jax.ShapeDtypeStruct((N_NODES, 128), F32),
    )(agg3, r3, Wrel3.T, brel3.reshape(1, 128))
    return x3
